# Initial kernel scaffold; baseline (speedup 1.0000x reference)
import jax
import jax.numpy as jnp
from jax.experimental import pallas as pl


def kernel(inputs, token_table, pos_table):
    raise NotImplementedError("write your pallas kernel here")


# speedup vs baseline: 2.2784x; 2.2784x over previous
import functools

import jax
import jax.numpy as jnp
from jax import lax
from jax.experimental import pallas as pl
from jax.experimental.pallas import tpu as pltpu
from jax.experimental.pallas import tpu_sc as plsc

CHUNK = 128
LANES = 16


def kernel(inputs, token_table, pos_table):
    B, L = inputs.shape
    V, D = token_table.shape
    flat = B * L
    idx = inputs.reshape(flat // CHUNK, CHUNK).astype(jnp.int32)

    info = plsc.get_sparse_core_info()
    NC, NS = info.num_cores, info.num_subcores
    NW = NC * NS
    chunks_per_w = flat // (NW * CHUNK)
    mesh = plsc.VectorSubcoreMesh(core_axis_name="c", subcore_axis_name="s")

    @functools.partial(
        pl.kernel,
        mesh=mesh,
        compiler_params=pltpu.CompilerParams(use_tc_tiling_on_sc=False),
        out_type=jax.ShapeDtypeStruct((flat, D), jnp.float32),
        scratch_types=[
            pltpu.VMEM((chunks_per_w, CHUNK), jnp.int32),
            pltpu.VMEM((CHUNK, D), jnp.float32),
            pltpu.VMEM((2 * L, D), jnp.float32),
            pltpu.SemaphoreType.DMA,
        ],
    )
    def k(idx_hbm, tok_hbm, pos_hbm, out_hbm, idx_v, rows_v, pos_v, sem):
        wid = lax.axis_index("s") * NC + lax.axis_index("c")
        chunk0 = wid * chunks_per_w
        pltpu.sync_copy(idx_hbm.at[pl.ds(chunk0, chunks_per_w)], idx_v)
        pltpu.sync_copy(pos_hbm, pos_v.at[pl.ds(0, L)])
        pltpu.sync_copy(pos_hbm, pos_v.at[pl.ds(L, L)])

        def chunk_body(g, carry):
            pltpu.async_copy(tok_hbm.at[idx_v.at[g]], rows_v, sem).wait()
            off = lax.rem(g * CHUNK, L)

            def row_body(r, c):
                for j in range(D // LANES):
                    sl = pl.ds(j * LANES, LANES)
                    rows_v[r, sl] = rows_v[r, sl] + pos_v[off + r, sl]
                return c

            lax.fori_loop(0, CHUNK, row_body, 0, unroll=2)
            pltpu.sync_copy(rows_v, out_hbm.at[pl.ds((chunk0 + g) * CHUNK, CHUNK)])
            return carry

        lax.fori_loop(0, chunks_per_w, chunk_body, 0)

    out = k(idx, token_table, pos_table)
    return out.reshape(B, L, D)


# speedup vs baseline: 2.7505x; 1.2072x over previous
import functools

import jax
import jax.numpy as jnp
from jax import lax
from jax.experimental import pallas as pl
from jax.experimental.pallas import tpu as pltpu
from jax.experimental.pallas import tpu_sc as plsc

CHUNK = 128
LANES = 16
NBUF = 2


def kernel(inputs, token_table, pos_table):
    B, L = inputs.shape
    V, D = token_table.shape
    flat = B * L
    idx = inputs.reshape(flat // CHUNK, CHUNK).astype(jnp.int32)

    info = plsc.get_sparse_core_info()
    NC, NS = info.num_cores, info.num_subcores
    NW = NC * NS
    chunks_per_w = flat // (NW * CHUNK)
    mesh = plsc.VectorSubcoreMesh(core_axis_name="c", subcore_axis_name="s")

    @functools.partial(
        pl.kernel,
        mesh=mesh,
        compiler_params=pltpu.CompilerParams(use_tc_tiling_on_sc=False),
        out_type=jax.ShapeDtypeStruct((flat, D), jnp.float32),
        scratch_types=[
            pltpu.VMEM((chunks_per_w, CHUNK), jnp.int32),
            pltpu.VMEM((CHUNK, D), jnp.float32),
            pltpu.VMEM((CHUNK, D), jnp.float32),
            pltpu.VMEM((CHUNK, D), jnp.float32),
            pltpu.VMEM((CHUNK, D), jnp.float32),
            pltpu.VMEM((2 * L, D), jnp.float32),
            pltpu.SemaphoreType.DMA,
            pltpu.SemaphoreType.DMA,
            pltpu.SemaphoreType.DMA,
            pltpu.SemaphoreType.DMA,
        ],
    )
    def k(idx_hbm, tok_hbm, pos_hbm, out_hbm,
          idx_v, gb0, gb1, ob0, ob1, pos_v, gs0, gs1, os0, os1):
        gbufs, obufs = (gb0, gb1), (ob0, ob1)
        gsems, osems = (gs0, gs1), (os0, os1)
        wid = lax.axis_index("s") * NC + lax.axis_index("c")
        chunk0 = wid * chunks_per_w
        pltpu.sync_copy(idx_hbm.at[pl.ds(chunk0, chunks_per_w)], idx_v)
        pltpu.sync_copy(pos_hbm, pos_v.at[pl.ds(0, L)])
        pltpu.sync_copy(pos_hbm, pos_v.at[pl.ds(L, L)])

        def gather_start(g, b):
            pltpu.async_copy(tok_hbm.at[idx_v.at[g]], gbufs[b], gsems[b])

        def gather_wait(g, b):
            pltpu.make_async_copy(tok_hbm.at[idx_v.at[g]], gbufs[b], gsems[b]).wait()

        def out_start(g, b):
            pltpu.async_copy(
                obufs[b], out_hbm.at[pl.ds((chunk0 + g) * CHUNK, CHUNK)], osems[b])

        def out_wait(g, b):
            pltpu.make_async_copy(
                obufs[b], out_hbm.at[pl.ds((chunk0 + g) * CHUNK, CHUNK)],
                osems[b]).wait()

        for b in range(NBUF):
            gather_start(b, b)

        n_iter = chunks_per_w // NBUF

        def iter_body(i, carry):
            for b in range(NBUF):
                g = i * NBUF + b
                gather_wait(g, b)

                @pl.when(i >= 1)
                def _():
                    out_wait(g - NBUF, b)

                off = lax.rem(g * CHUNK, L)
                gbuf, obuf = gbufs[b], obufs[b]

                def row_body(r, c):
                    for j in range(D // LANES):
                        sl = pl.ds(j * LANES, LANES)
                        obuf[r, sl] = gbuf[r, sl] + pos_v[off + r, sl]
                    return c

                lax.fori_loop(0, CHUNK, row_body, 0, unroll=4)
                out_start(g, b)

                @pl.when(i < n_iter - 1)
                def _():
                    gather_start(g + NBUF, b)

            return carry

        lax.fori_loop(0, n_iter, iter_body, 0)
        for b in range(NBUF):
            out_wait(chunks_per_w - NBUF + b, b)

    out = k(idx, token_table, pos_table)
    return out.reshape(B, L, D)
